# baseline (speedup 1.0000x reference)
import functools

import jax
import jax.numpy as jnp
from jax import lax
from jax.experimental import pallas as pl
from jax.experimental.pallas import tpu as pltpu
from jax.experimental.pallas import tpu_sc as plsc

_CH = 128
_GB = 2


@functools.cache
def _make_sc_gather(B0: int, B1: int, D: int):
    info = plsc.get_sparse_core_info()
    NC, NS, L = info.num_cores, info.num_subcores, info.num_lanes
    NW = NC * NS
    NBC = B0 // _CH
    NGS = NBC // _GB
    NG = B1 * NGS
    assert NG % NW == 0
    GPW = NG // NW
    DT = D // 8
    ZROWS = B1 * DT * NBC * 8
    mesh = plsc.VectorSubcoreMesh(core_axis_name="c", subcore_axis_name="s")

    @functools.partial(
        pl.kernel,
        mesh=mesh,
        out_type=jax.ShapeDtypeStruct((ZROWS, 128), jnp.float32),
        compiler_params=pltpu.CompilerParams(
            use_tc_tiling_on_sc=False, needs_layout_passes=False),
        scratch_types=[
            pltpu.VMEM((GPW * _GB, _CH), jnp.int32),
            pltpu.VMEM((2, _GB * _CH, D), jnp.float32),
            pltpu.VMEM((2, _GB * 8 * DT, 128), jnp.float32),
            pltpu.SemaphoreType.DMA((2,)),
            pltpu.SemaphoreType.DMA((2,)),
        ],
    )
    def gather_kernel(idx_hbm, table_hbm, z_hbm, idx_v, rows_v, zbuf_v, gsem, osem):
        wid = lax.axis_index("s") * NC + lax.axis_index("c")
        g0 = wid * GPW
        pltpu.sync_copy(idx_hbm.at[pl.ds(g0 * _GB, GPW * _GB)], idx_v)

        def fire_gathers(l, b):
            for k in range(_GB):
                pltpu.async_copy(
                    table_hbm.at[idx_v.at[_GB * l + k]],
                    rows_v.at[b, pl.ds(k * _CH, _CH), :],
                    gsem.at[b],
                )

        def wait_gathers(l, b):
            for k in range(_GB):
                pltpu.make_async_copy(
                    table_hbm.at[idx_v.at[_GB * l + k]],
                    rows_v.at[b, pl.ds(k * _CH, _CH), :],
                    gsem.at[b],
                ).wait()

        def row_base(l):
            g = g0 + l
            s = g // NGS
            gb = g - s * NGS
            return s * (DT * NBC * 8) + gb * (_GB * 8)

        def fire_stores(l, b):
            base = row_base(l)
            for dt in range(DT):
                pltpu.async_copy(
                    zbuf_v.at[b, pl.ds(dt * _GB * 8, _GB * 8), :],
                    z_hbm.at[pl.ds(base + dt * NBC * 8, _GB * 8)],
                    osem.at[b],
                )

        def wait_stores(l, b):
            base = row_base(l)
            for dt in range(DT):
                pltpu.make_async_copy(
                    zbuf_v.at[b, pl.ds(dt * _GB * 8, _GB * 8), :],
                    z_hbm.at[pl.ds(base + dt * NBC * 8, _GB * 8)],
                    osem.at[b],
                ).wait()

        iota = lax.iota(jnp.int32, L)
        row_ids = [[(k * _CH + jv * L) + iota for jv in range(_CH // L)]
                   for k in range(_GB)]

        def transpose(b):
            rows = rows_v.at[b]
            zb = zbuf_v.at[b]

            def dt_body(dt, carry):
                for dsub in range(8):
                    d = 8 * dt + dsub
                    dvec = iota * 0 + d
                    for k in range(_GB):
                        zr = dt * (_GB * 8) + k * 8 + dsub
                        for jv in range(_CH // L):
                            v = plsc.load_gather(rows, [row_ids[k][jv], dvec])
                            zb[zr, pl.ds(jv * L, L)] = v
                return carry

            lax.fori_loop(0, DT, dt_body, 0)

        def group_body(l, p):
            wait_gathers(l, p)

            @pl.when(l >= 2)
            def _drain():
                wait_stores(l - 2, p)

            transpose(p)
            fire_stores(l, p)

            @pl.when(l + 2 < GPW)
            def _prefetch():
                fire_gathers(l + 2, p)

        fire_gathers(0, 0)
        fire_gathers(1, 1)

        def body(i, carry):
            group_body(2 * i, 0)
            group_body(2 * i + 1, 1)
            return carry

        lax.fori_loop(0, GPW // 2, body, 0)
        for p in range(2):
            wait_stores(GPW - 2 + p, p)

    return gather_kernel


def kernel(pre_embedding, table):
    B0, B1 = pre_embedding.shape
    V, D = table.shape
    idx2d = pre_embedding.T.reshape(B1 * (B0 // _CH), _CH).astype(jnp.int32)
    z = _make_sc_gather(B0, B1, D)(idx2d, table)
    z5 = z.reshape(B1, D // 8, B0 // _CH, 8, _CH)
    return z5.transpose(2, 4, 0, 1, 3).reshape(B0, B1, D)
